# baseline (speedup 1.0000x reference)
import functools

import jax
import jax.numpy as jnp
from jax import lax
from jax.experimental import pallas as pl
from jax.experimental.pallas import tpu as pltpu
from jax.experimental.pallas import tpu_sc as plsc

NC = 2
NS = 16
NW = NC * NS
LANES = 16
B = 128


def _sc_mesh():
    return plsc.VectorSubcoreMesh(core_axis_name="c", subcore_axis_name="s")


def _make_deg_kernel(n_pad, ch):
    rt = n_pad // NS

    @functools.partial(
        pl.kernel,
        out_type=jax.ShapeDtypeStruct((NC, n_pad, LANES), jnp.float32),
        mesh=_sc_mesh(),
        scratch_types=[
            pltpu.VMEM((ch, B), jnp.int32),
            pltpu.VMEM((n_pad,), jnp.float32),
            pltpu.VMEM_SHARED((NS, n_pad), jnp.float32),
            pltpu.VMEM((NS, rt), jnp.float32),
            pltpu.VMEM((rt, LANES), jnp.float32),
        ],
        compiler_params=pltpu.CompilerParams(needs_layout_passes=False),
    )
    def deg_kernel(src_hbm, out_hbm, idx_v, hist_v, stage_sh, colsum_v, bc_v):
        c = lax.axis_index("c")
        s = lax.axis_index("s")
        w = c * NS + s
        zero16 = jnp.zeros((LANES,), jnp.float32)

        def zbody(i, carry):
            hist_v[pl.ds(i * LANES, LANES)] = zero16
            return carry

        lax.fori_loop(0, n_pad // LANES, zbody, 0)
        pltpu.sync_copy(src_hbm.at[w], idx_v)

        def hbody(j, carry):
            for k in range(B // LANES):
                v = idx_v[j, pl.ds(k * LANES, LANES)]
                plsc.addupdate_scatter(hist_v, [v], jnp.ones((LANES,), jnp.float32))
            return carry

        lax.fori_loop(0, ch, hbody, 0)

        pltpu.sync_copy(hist_v, stage_sh.at[s])
        plsc.subcore_barrier()
        for r in range(NS):
            pltpu.sync_copy(stage_sh.at[r, pl.ds(s * rt, rt)], colsum_v.at[r])

        def rbody(i, carry):
            acc = colsum_v[0, pl.ds(i * LANES, LANES)]
            for r in range(1, NS):
                acc = acc + colsum_v[r, pl.ds(i * LANES, LANES)]
            for l in range(LANES):
                bc_v[i * LANES + l, :] = jnp.full((LANES,), acc[l], jnp.float32)
            return carry

        lax.fori_loop(0, rt // LANES, rbody, 0)
        pltpu.sync_copy(bc_v, out_hbm.at[c, pl.ds(s * rt, rt)])

    return deg_kernel


def _make_prop_kernel(n_pad, ch, d):
    rt = n_pad // NS
    bp = B // 2
    cw = 16
    rb = 4
    assert ch % cw == 0 and cw % 2 == 0
    ncc = ch // cw

    @functools.partial(
        pl.kernel,
        out_type=jax.ShapeDtypeStruct((NC, n_pad, d), jnp.float32),
        mesh=_sc_mesh(),
        scratch_types=[
            pltpu.VMEM((2, cw, bp), jnp.int32),
            pltpu.VMEM((2, cw, bp), jnp.int32),
            pltpu.VMEM((rb, bp, d), jnp.float32),
            pltpu.VMEM((32, d), jnp.float32),
            pltpu.VMEM_SHARED((n_pad, d), jnp.float32),
            pltpu.SemaphoreType.DMA,
            [pltpu.SemaphoreType.DMA] * rb,
            [pltpu.SemaphoreType.DMA] * rb,
        ],
    )
    def prop_kernel(u_hbm, srcg_hbm, dstp_hbm, out_hbm,
                    src_cv, dst_cv, rows_v, zb_v, acc_sh, semi, gsem, ssem):
        c = lax.axis_index("c")
        s = lax.axis_index("s")
        w = c * NS + s
        pltpu.async_copy(srcg_hbm.at[w, pl.ds(0, cw)], src_cv.at[0], semi)
        pltpu.async_copy(dstp_hbm.at[w, pl.ds(0, cw)], dst_cv.at[0], semi)
        zero16 = jnp.zeros((LANES,), jnp.float32)

        def zfill(i, carry):
            for k in range(d // LANES):
                zb_v[i, pl.ds(k * LANES, LANES)] = zero16
            return carry

        lax.fori_loop(0, 32, zfill, 0)
        assert rt % 32 == 0
        for i in range(rt // 32):
            pltpu.sync_copy(zb_v, acc_sh.at[pl.ds(s * rt + i * 32, 32)])
        plsc.subcore_barrier()

        def chunk_body(cc, carry):
            bsel = lax.rem(cc, 2)
            nb = lax.rem(cc + 1, 2)
            pltpu.make_async_copy(
                srcg_hbm.at[w, pl.ds(cc * cw, cw)], src_cv.at[bsel], semi).wait()
            pltpu.make_async_copy(
                dstp_hbm.at[w, pl.ds(cc * cw, cw)], dst_cv.at[bsel], semi).wait()

            @pl.when(cc + 1 < ncc)
            def _():
                pltpu.async_copy(
                    srcg_hbm.at[w, pl.ds((cc + 1) * cw, cw)], src_cv.at[nb], semi)
                pltpu.async_copy(
                    dstp_hbm.at[w, pl.ds((cc + 1) * cw, cw)], dst_cv.at[nb], semi)

            sv = src_cv.at[bsel]
            dv = dst_cv.at[bsel]
            svn = src_cv.at[nb]

            for j in range(cw):
                k = j % rb
                k2 = (j + 2) % rb
                if j < 2:
                    @pl.when(cc == 0)
                    def _():
                        pltpu.async_copy(u_hbm.at[sv.at[j]], rows_v.at[k], gsem[k])
                pltpu.make_async_copy(u_hbm.at[sv.at[j]], rows_v.at[k], gsem[k]).wait()
                pltpu.async_copy(rows_v.at[k], acc_sh.at[dv.at[j]], ssem[k], add=True)
                if j < 2:
                    @pl.when(cc > 0)
                    def _():
                        pltpu.make_async_copy(
                            rows_v.at[k2], acc_sh.at[dv.at[j]], ssem[k2]).wait()
                else:
                    pltpu.make_async_copy(
                        rows_v.at[k2], acc_sh.at[dv.at[j]], ssem[k2]).wait()
                if j < cw - 2:
                    pltpu.async_copy(u_hbm.at[sv.at[j + 2]], rows_v.at[k2], gsem[k2])
                else:
                    @pl.when(cc + 1 < ncc)
                    def _():
                        pltpu.async_copy(
                            u_hbm.at[svn.at[j - (cw - 2)]], rows_v.at[k2], gsem[k2])
            return carry

        lax.fori_loop(0, ncc, chunk_body, 0)
        dvl = dst_cv.at[(ncc - 1) % 2]
        pltpu.make_async_copy(
            rows_v.at[(cw - 2) % rb], acc_sh.at[dvl.at[cw - 2]], ssem[(cw - 2) % rb]).wait()
        pltpu.make_async_copy(
            rows_v.at[(cw - 1) % rb], acc_sh.at[dvl.at[cw - 1]], ssem[(cw - 1) % rb]).wait()
        plsc.subcore_barrier()
        pltpu.sync_copy(acc_sh.at[pl.ds(s * rt, rt)], out_hbm.at[c, pl.ds(s * rt, rt)])

    return prop_kernel


def _dis_from_degp(degp_ref):
    deg = degp_ref[0, :, 0:1] + degp_ref[1, :, 0:1]
    safe = jnp.where(deg > 0.0, deg, 1.0)
    return jnp.where(deg > 0.0, lax.rsqrt(safe), 0.0)


def _tc_pre(degp, x, w1, n_pad, blk, d):
    def body(degp_ref, x_ref, w_ref, o_ref):
        dis = _dis_from_degp(degp_ref)
        o_ref[...] = jnp.dot(x_ref[...], w_ref[...],
                             preferred_element_type=jnp.float32) * dis

    return pl.pallas_call(
        body,
        grid=(n_pad // blk,),
        in_specs=[
            pl.BlockSpec((NC, blk, LANES), lambda i: (0, i, 0)),
            pl.BlockSpec((blk, d), lambda i: (i, 0)),
            pl.BlockSpec((d, d), lambda i: (0, 0)),
        ],
        out_specs=pl.BlockSpec((blk, d), lambda i: (i, 0)),
        out_shape=jax.ShapeDtypeStruct((n_pad, d), jnp.float32),
    )(degp, x, w1)


def _tc_mid(degp, x, s1p, w0, b, w1, n_pad, blk, d):
    def body(degp_ref, x_ref, sp_ref, w0_ref, b_ref, w1_ref, h_ref, u_ref):
        dis = _dis_from_degp(degp_ref)
        s_sum = sp_ref[0] + sp_ref[1]
        h = jnp.maximum(
            jnp.dot(x_ref[...], w0_ref[...], preferred_element_type=jnp.float32)
            - dis * s_sum + b_ref[...], 0.0)
        h_ref[...] = h
        u_ref[...] = jnp.dot(h, w1_ref[...],
                             preferred_element_type=jnp.float32) * dis

    return pl.pallas_call(
        body,
        grid=(n_pad // blk,),
        in_specs=[
            pl.BlockSpec((NC, blk, LANES), lambda i: (0, i, 0)),
            pl.BlockSpec((blk, d), lambda i: (i, 0)),
            pl.BlockSpec((NC, blk, d), lambda i: (0, i, 0)),
            pl.BlockSpec((d, d), lambda i: (0, 0)),
            pl.BlockSpec((1, d), lambda i: (0, 0)),
            pl.BlockSpec((d, d), lambda i: (0, 0)),
        ],
        out_specs=[
            pl.BlockSpec((blk, d), lambda i: (i, 0)),
            pl.BlockSpec((blk, d), lambda i: (i, 0)),
        ],
        out_shape=[
            jax.ShapeDtypeStruct((n_pad, d), jnp.float32),
            jax.ShapeDtypeStruct((n_pad, d), jnp.float32),
        ],
    )(degp, x, s1p, w0, b, w1)


def _tc_post(degp, h, s2p, w0, b, wfc_row, bfc2, n_pad, blk, d):
    def body(degp_ref, h_ref, sp_ref, w0_ref, b_ref, wfc_ref, bfc_ref,
             emb_ref, out_ref):
        dis = _dis_from_degp(degp_ref)
        s_sum = sp_ref[0] + sp_ref[1]
        emb = (jnp.dot(h_ref[...], w0_ref[...], preferred_element_type=jnp.float32)
               - dis * s_sum + b_ref[...])
        emb_ref[...] = emb
        logits = (jnp.sum(emb * wfc_ref[...], axis=1, keepdims=True)
                  + bfc_ref[:, 0:1])
        out_ref[...] = jax.nn.sigmoid(logits)

    return pl.pallas_call(
        body,
        grid=(n_pad // blk,),
        in_specs=[
            pl.BlockSpec((NC, blk, LANES), lambda i: (0, i, 0)),
            pl.BlockSpec((blk, d), lambda i: (i, 0)),
            pl.BlockSpec((NC, blk, d), lambda i: (0, i, 0)),
            pl.BlockSpec((d, d), lambda i: (0, 0)),
            pl.BlockSpec((1, d), lambda i: (0, 0)),
            pl.BlockSpec((1, d), lambda i: (0, 0)),
            pl.BlockSpec((1, 1), lambda i: (0, 0)),
        ],
        out_specs=[
            pl.BlockSpec((blk, d), lambda i: (i, 0)),
            pl.BlockSpec((blk, 1), lambda i: (i, 0)),
        ],
        out_shape=[
            jax.ShapeDtypeStruct((n_pad, d), jnp.float32),
            jax.ShapeDtypeStruct((n_pad, 1), jnp.float32),
        ],
    )(degp, h, s2p, w0, b, wfc_row, bfc2)


def kernel(x, edge_index, W0a, W1a, ba, W0b, W1b, bb, Wfc, bfc):
    n, d = x.shape
    e = edge_index.shape[1]
    ch_deg = -(-e // (NW * B))
    ch_deg += (-ch_deg) % 2
    ch = 2 * ch_deg
    ch += (-ch) % 32
    ch_deg = ch // 2
    e_pad = NW * ch_deg * B
    n_pad = -(-n // (NS * LANES)) * (NS * LANES)
    if n_pad < n + 8:
        n_pad += NS * LANES
    blk = max(b for b in range(8, min(n, 2048) + 1, 8) if n % b == 0)

    src = edge_index[0]
    dst = edge_index[1]
    pad = jnp.arange(e_pad - e, dtype=jnp.int32) % 8
    src_g = jnp.concatenate([src, pad]).reshape(NW, ch, B // 2)
    src_d = jnp.concatenate([src, n + pad]).reshape(NW, ch_deg, B)
    dst_p = jnp.concatenate([dst, n + pad]).reshape(NW, ch, B // 2)
    degp = _make_deg_kernel(n_pad, ch_deg)(src_d)

    prop = _make_prop_kernel(n_pad, ch, d)
    u1 = _tc_pre(degp, x, W1a, n, blk, d)
    s1p = prop(u1, src_g, dst_p)
    h, u2 = _tc_mid(degp, x, s1p, W0a, ba.reshape(1, d), W1b, n, blk, d)
    s2p = prop(u2, src_g, dst_p)
    emb, out = _tc_post(degp, h, s2p, W0b, bb.reshape(1, d),
                        Wfc.reshape(1, d), bfc.reshape(1, 1), n, blk, d)
    return (out, emb)


# speedup vs baseline: 1.0466x; 1.0466x over previous
import functools

import jax
import jax.numpy as jnp
from jax import lax
from jax.experimental import pallas as pl
from jax.experimental.pallas import tpu as pltpu
from jax.experimental.pallas import tpu_sc as plsc

NC = 2
NS = 16
NW = NC * NS
LANES = 16
B = 128


def _sc_mesh():
    return plsc.VectorSubcoreMesh(core_axis_name="c", subcore_axis_name="s")


def _make_deg_kernel(n_pad, ch):
    rt = n_pad // NS

    @functools.partial(
        pl.kernel,
        out_type=jax.ShapeDtypeStruct((NC, n_pad, LANES), jnp.float32),
        mesh=_sc_mesh(),
        scratch_types=[
            pltpu.VMEM((ch, B), jnp.int32),
            pltpu.VMEM((n_pad,), jnp.float32),
            pltpu.VMEM_SHARED((NS, n_pad), jnp.float32),
            pltpu.VMEM((NS, rt), jnp.float32),
            pltpu.VMEM((rt, LANES), jnp.float32),
        ],
        compiler_params=pltpu.CompilerParams(needs_layout_passes=False),
    )
    def deg_kernel(src_hbm, out_hbm, idx_v, hist_v, stage_sh, colsum_v, bc_v):
        c = lax.axis_index("c")
        s = lax.axis_index("s")
        w = c * NS + s
        zero16 = jnp.zeros((LANES,), jnp.float32)

        def zbody(i, carry):
            hist_v[pl.ds(i * LANES, LANES)] = zero16
            return carry

        lax.fori_loop(0, n_pad // LANES, zbody, 0)
        pltpu.sync_copy(src_hbm.at[w], idx_v)

        def hbody(j, carry):
            for k in range(B // LANES):
                v = idx_v[j, pl.ds(k * LANES, LANES)]
                plsc.addupdate_scatter(hist_v, [v], jnp.ones((LANES,), jnp.float32))
            return carry

        lax.fori_loop(0, ch, hbody, 0)

        pltpu.sync_copy(hist_v, stage_sh.at[s])
        plsc.subcore_barrier()
        for r in range(NS):
            pltpu.sync_copy(stage_sh.at[r, pl.ds(s * rt, rt)], colsum_v.at[r])

        def rbody(i, carry):
            acc = colsum_v[0, pl.ds(i * LANES, LANES)]
            for r in range(1, NS):
                acc = acc + colsum_v[r, pl.ds(i * LANES, LANES)]
            for l in range(LANES):
                bc_v[i * LANES + l, :] = jnp.full((LANES,), acc[l], jnp.float32)
            return carry

        lax.fori_loop(0, rt // LANES, rbody, 0)
        pltpu.sync_copy(bc_v, out_hbm.at[c, pl.ds(s * rt, rt)])

    return deg_kernel


def _make_prop_kernel(n_pad, ch, d):
    rt = n_pad // NS
    bp = B // 2
    cw = 16
    rb = 4
    assert ch % cw == 0 and cw % 2 == 0
    ncc = ch // cw

    @functools.partial(
        pl.kernel,
        out_type=jax.ShapeDtypeStruct((NC, n_pad, d), jnp.float32),
        mesh=_sc_mesh(),
        scratch_types=[
            pltpu.VMEM((2, cw, bp), jnp.int32),
            pltpu.VMEM((2, cw, bp), jnp.int32),
            pltpu.VMEM((rb, bp, d), jnp.float32),
            pltpu.VMEM_SHARED((n_pad, d), jnp.float32),
            pltpu.SemaphoreType.DMA,
            [pltpu.SemaphoreType.DMA] * rb,
            [pltpu.SemaphoreType.DMA] * rb,
        ],
    )
    def prop_kernel(u_hbm, srcg_hbm, dstp_hbm, zeros_hbm, out_hbm,
                    src_cv, dst_cv, rows_v, acc_sh, semi, gsem, ssem):
        c = lax.axis_index("c")
        s = lax.axis_index("s")
        w = c * NS + s
        pltpu.async_copy(srcg_hbm.at[w, pl.ds(0, cw)], src_cv.at[0], semi)
        pltpu.async_copy(dstp_hbm.at[w, pl.ds(0, cw)], dst_cv.at[0], semi)
        pltpu.sync_copy(zeros_hbm.at[pl.ds(s * rt, rt)], acc_sh.at[pl.ds(s * rt, rt)])
        plsc.subcore_barrier()

        def chunk_body(cc, carry):
            bsel = lax.rem(cc, 2)
            nb = lax.rem(cc + 1, 2)
            pltpu.make_async_copy(
                srcg_hbm.at[w, pl.ds(cc * cw, cw)], src_cv.at[bsel], semi).wait()
            pltpu.make_async_copy(
                dstp_hbm.at[w, pl.ds(cc * cw, cw)], dst_cv.at[bsel], semi).wait()

            @pl.when(cc + 1 < ncc)
            def _():
                pltpu.async_copy(
                    srcg_hbm.at[w, pl.ds((cc + 1) * cw, cw)], src_cv.at[nb], semi)
                pltpu.async_copy(
                    dstp_hbm.at[w, pl.ds((cc + 1) * cw, cw)], dst_cv.at[nb], semi)

            sv = src_cv.at[bsel]
            dv = dst_cv.at[bsel]
            svn = src_cv.at[nb]

            for j in range(cw):
                k = j % rb
                k2 = (j + 2) % rb
                if j < 2:
                    @pl.when(cc == 0)
                    def _():
                        pltpu.async_copy(u_hbm.at[sv.at[j]], rows_v.at[k], gsem[k])
                pltpu.make_async_copy(u_hbm.at[sv.at[j]], rows_v.at[k], gsem[k]).wait()
                pltpu.async_copy(rows_v.at[k], acc_sh.at[dv.at[j]], ssem[k], add=True)
                if j < 2:
                    @pl.when(cc > 0)
                    def _():
                        pltpu.make_async_copy(
                            rows_v.at[k2], acc_sh.at[dv.at[j]], ssem[k2]).wait()
                else:
                    pltpu.make_async_copy(
                        rows_v.at[k2], acc_sh.at[dv.at[j]], ssem[k2]).wait()
                if j < cw - 2:
                    pltpu.async_copy(u_hbm.at[sv.at[j + 2]], rows_v.at[k2], gsem[k2])
                else:
                    @pl.when(cc + 1 < ncc)
                    def _():
                        pltpu.async_copy(
                            u_hbm.at[svn.at[j - (cw - 2)]], rows_v.at[k2], gsem[k2])
            return carry

        lax.fori_loop(0, ncc, chunk_body, 0)
        dvl = dst_cv.at[(ncc - 1) % 2]
        pltpu.make_async_copy(
            rows_v.at[(cw - 2) % rb], acc_sh.at[dvl.at[cw - 2]], ssem[(cw - 2) % rb]).wait()
        pltpu.make_async_copy(
            rows_v.at[(cw - 1) % rb], acc_sh.at[dvl.at[cw - 1]], ssem[(cw - 1) % rb]).wait()
        plsc.subcore_barrier()
        pltpu.sync_copy(acc_sh.at[pl.ds(s * rt, rt)], out_hbm.at[c, pl.ds(s * rt, rt)])

    return prop_kernel


def _dis_from_degp(degp_ref):
    deg = degp_ref[0, :, 0:1] + degp_ref[1, :, 0:1]
    safe = jnp.where(deg > 0.0, deg, 1.0)
    return jnp.where(deg > 0.0, lax.rsqrt(safe), 0.0)


def _tc_pre(degp, x, w1, n_pad, blk, d):
    def body(degp_ref, x_ref, w_ref, o_ref):
        dis = _dis_from_degp(degp_ref)
        o_ref[...] = jnp.dot(x_ref[...], w_ref[...],
                             preferred_element_type=jnp.float32) * dis

    return pl.pallas_call(
        body,
        grid=(n_pad // blk,),
        in_specs=[
            pl.BlockSpec((NC, blk, LANES), lambda i: (0, i, 0)),
            pl.BlockSpec((blk, d), lambda i: (i, 0)),
            pl.BlockSpec((d, d), lambda i: (0, 0)),
        ],
        out_specs=pl.BlockSpec((blk, d), lambda i: (i, 0)),
        out_shape=jax.ShapeDtypeStruct((n_pad, d), jnp.float32),
    )(degp, x, w1)


def _tc_mid(degp, x, s1p, w0, b, w1, n_pad, blk, d):
    def body(degp_ref, x_ref, sp_ref, w0_ref, b_ref, w1_ref, h_ref, u_ref):
        dis = _dis_from_degp(degp_ref)
        s_sum = sp_ref[0] + sp_ref[1]
        h = jnp.maximum(
            jnp.dot(x_ref[...], w0_ref[...], preferred_element_type=jnp.float32)
            - dis * s_sum + b_ref[...], 0.0)
        h_ref[...] = h
        u_ref[...] = jnp.dot(h, w1_ref[...],
                             preferred_element_type=jnp.float32) * dis

    return pl.pallas_call(
        body,
        grid=(n_pad // blk,),
        in_specs=[
            pl.BlockSpec((NC, blk, LANES), lambda i: (0, i, 0)),
            pl.BlockSpec((blk, d), lambda i: (i, 0)),
            pl.BlockSpec((NC, blk, d), lambda i: (0, i, 0)),
            pl.BlockSpec((d, d), lambda i: (0, 0)),
            pl.BlockSpec((1, d), lambda i: (0, 0)),
            pl.BlockSpec((d, d), lambda i: (0, 0)),
        ],
        out_specs=[
            pl.BlockSpec((blk, d), lambda i: (i, 0)),
            pl.BlockSpec((blk, d), lambda i: (i, 0)),
        ],
        out_shape=[
            jax.ShapeDtypeStruct((n_pad, d), jnp.float32),
            jax.ShapeDtypeStruct((n_pad, d), jnp.float32),
        ],
    )(degp, x, s1p, w0, b, w1)


def _tc_post(degp, h, s2p, w0, b, wfc_row, bfc2, n_pad, blk, d):
    def body(degp_ref, h_ref, sp_ref, w0_ref, b_ref, wfc_ref, bfc_ref,
             emb_ref, out_ref):
        dis = _dis_from_degp(degp_ref)
        s_sum = sp_ref[0] + sp_ref[1]
        emb = (jnp.dot(h_ref[...], w0_ref[...], preferred_element_type=jnp.float32)
               - dis * s_sum + b_ref[...])
        emb_ref[...] = emb
        logits = (jnp.sum(emb * wfc_ref[...], axis=1, keepdims=True)
                  + bfc_ref[:, 0:1])
        out_ref[...] = jax.nn.sigmoid(logits)

    return pl.pallas_call(
        body,
        grid=(n_pad // blk,),
        in_specs=[
            pl.BlockSpec((NC, blk, LANES), lambda i: (0, i, 0)),
            pl.BlockSpec((blk, d), lambda i: (i, 0)),
            pl.BlockSpec((NC, blk, d), lambda i: (0, i, 0)),
            pl.BlockSpec((d, d), lambda i: (0, 0)),
            pl.BlockSpec((1, d), lambda i: (0, 0)),
            pl.BlockSpec((1, d), lambda i: (0, 0)),
            pl.BlockSpec((1, 1), lambda i: (0, 0)),
        ],
        out_specs=[
            pl.BlockSpec((blk, d), lambda i: (i, 0)),
            pl.BlockSpec((blk, 1), lambda i: (i, 0)),
        ],
        out_shape=[
            jax.ShapeDtypeStruct((n_pad, d), jnp.float32),
            jax.ShapeDtypeStruct((n_pad, 1), jnp.float32),
        ],
    )(degp, h, s2p, w0, b, wfc_row, bfc2)


def kernel(x, edge_index, W0a, W1a, ba, W0b, W1b, bb, Wfc, bfc):
    n, d = x.shape
    e = edge_index.shape[1]
    ch_deg = -(-e // (NW * B))
    ch_deg += (-ch_deg) % 2
    ch = 2 * ch_deg
    ch += (-ch) % 32
    ch_deg = ch // 2
    e_pad = NW * ch_deg * B
    n_pad = -(-n // (NS * LANES)) * (NS * LANES)
    if n_pad < n + 8:
        n_pad += NS * LANES
    blk = max(b for b in range(8, min(n, 2048) + 1, 8) if n % b == 0)

    src = edge_index[0]
    dst = edge_index[1]
    pad = jnp.arange(e_pad - e, dtype=jnp.int32) % 8
    src_g = jnp.concatenate([src, pad]).reshape(NW, ch, B // 2)
    src_d = jnp.concatenate([src, n + pad]).reshape(NW, ch_deg, B)
    dst_p = jnp.concatenate([dst, n + pad]).reshape(NW, ch, B // 2)
    zeros_row = jnp.zeros((n_pad, d), jnp.float32)

    degp = _make_deg_kernel(n_pad, ch_deg)(src_d)

    prop = _make_prop_kernel(n_pad, ch, d)
    u1 = _tc_pre(degp, x, W1a, n, blk, d)
    s1p = prop(u1, src_g, dst_p, zeros_row)
    h, u2 = _tc_mid(degp, x, s1p, W0a, ba.reshape(1, d), W1b, n, blk, d)
    s2p = prop(u2, src_g, dst_p, zeros_row)
    emb, out = _tc_post(degp, h, s2p, W0b, bb.reshape(1, d),
                        Wfc.reshape(1, d), bfc.reshape(1, 1), n, blk, d)
    return (out, emb)


# speedup vs baseline: 1.0571x; 1.0100x over previous
import functools

import jax
import jax.numpy as jnp
from jax import lax
from jax.experimental import pallas as pl
from jax.experimental.pallas import tpu as pltpu
from jax.experimental.pallas import tpu_sc as plsc

NC = 2
NS = 16
NW = NC * NS
LANES = 16
B = 128


def _sc_mesh():
    return plsc.VectorSubcoreMesh(core_axis_name="c", subcore_axis_name="s")


def _make_deg_kernel(n_pad, ch):
    rt = n_pad // NS

    @functools.partial(
        pl.kernel,
        out_type=jax.ShapeDtypeStruct((NC, n_pad, LANES), jnp.float32),
        mesh=_sc_mesh(),
        scratch_types=[
            pltpu.VMEM((ch, B), jnp.int32),
            pltpu.VMEM((n_pad,), jnp.float32),
            pltpu.VMEM_SHARED((NS, n_pad), jnp.float32),
            pltpu.VMEM((NS, rt), jnp.float32),
            pltpu.VMEM((rt, LANES), jnp.float32),
        ],
        compiler_params=pltpu.CompilerParams(needs_layout_passes=False),
    )
    def deg_kernel(src_hbm, out_hbm, idx_v, hist_v, stage_sh, colsum_v, bc_v):
        c = lax.axis_index("c")
        s = lax.axis_index("s")
        w = c * NS + s
        zero16 = jnp.zeros((LANES,), jnp.float32)

        def zbody(i, carry):
            hist_v[pl.ds(i * LANES, LANES)] = zero16
            return carry

        lax.fori_loop(0, n_pad // LANES, zbody, 0)
        pltpu.sync_copy(src_hbm.at[w], idx_v)

        def hbody(j, carry):
            for k in range(B // LANES):
                v = idx_v[j, pl.ds(k * LANES, LANES)]
                plsc.addupdate_scatter(hist_v, [v], jnp.ones((LANES,), jnp.float32))
            return carry

        lax.fori_loop(0, ch, hbody, 0)

        pltpu.sync_copy(hist_v, stage_sh.at[s])
        plsc.subcore_barrier()
        for r in range(NS):
            pltpu.sync_copy(stage_sh.at[r, pl.ds(s * rt, rt)], colsum_v.at[r])

        def rbody(i, carry):
            acc = colsum_v[0, pl.ds(i * LANES, LANES)]
            for r in range(1, NS):
                acc = acc + colsum_v[r, pl.ds(i * LANES, LANES)]
            for l in range(LANES):
                bc_v[i * LANES + l, :] = jnp.full((LANES,), acc[l], jnp.float32)
            return carry

        lax.fori_loop(0, rt // LANES, rbody, 0)
        pltpu.sync_copy(bc_v, out_hbm.at[c, pl.ds(s * rt, rt)])

    return deg_kernel


def _make_prop_kernel(n_pad, ch, d):
    rt = n_pad // NS
    bp = 80
    cw = 16
    rb = 4
    assert ch % cw == 0 and cw % 2 == 0
    ncc = ch // cw

    @functools.partial(
        pl.kernel,
        out_type=jax.ShapeDtypeStruct((NC, n_pad, d), jnp.float32),
        mesh=_sc_mesh(),
        scratch_types=[
            pltpu.VMEM((2, cw, bp), jnp.int32),
            pltpu.VMEM((2, cw, bp), jnp.int32),
            pltpu.VMEM((rb, bp, d), jnp.float32),
            pltpu.VMEM_SHARED((n_pad, d), jnp.float32),
            pltpu.SemaphoreType.DMA,
            [pltpu.SemaphoreType.DMA] * rb,
            [pltpu.SemaphoreType.DMA] * rb,
        ],
    )
    def prop_kernel(u_hbm, srcg_hbm, dstp_hbm, zeros_hbm, out_hbm,
                    src_cv, dst_cv, rows_v, acc_sh, semi, gsem, ssem):
        c = lax.axis_index("c")
        s = lax.axis_index("s")
        w = c * NS + s
        pltpu.async_copy(srcg_hbm.at[w, pl.ds(0, cw)], src_cv.at[0], semi)
        pltpu.async_copy(dstp_hbm.at[w, pl.ds(0, cw)], dst_cv.at[0], semi)
        pltpu.sync_copy(zeros_hbm.at[pl.ds(s * rt, rt)], acc_sh.at[pl.ds(s * rt, rt)])
        plsc.subcore_barrier()

        def chunk_body(cc, carry):
            bsel = lax.rem(cc, 2)
            nb = lax.rem(cc + 1, 2)
            pltpu.make_async_copy(
                srcg_hbm.at[w, pl.ds(cc * cw, cw)], src_cv.at[bsel], semi).wait()
            pltpu.make_async_copy(
                dstp_hbm.at[w, pl.ds(cc * cw, cw)], dst_cv.at[bsel], semi).wait()

            @pl.when(cc + 1 < ncc)
            def _():
                pltpu.async_copy(
                    srcg_hbm.at[w, pl.ds((cc + 1) * cw, cw)], src_cv.at[nb], semi)
                pltpu.async_copy(
                    dstp_hbm.at[w, pl.ds((cc + 1) * cw, cw)], dst_cv.at[nb], semi)

            sv = src_cv.at[bsel]
            dv = dst_cv.at[bsel]
            svn = src_cv.at[nb]

            for j in range(cw):
                k = j % rb
                k2 = (j + 2) % rb
                if j < 2:
                    @pl.when(cc == 0)
                    def _():
                        pltpu.async_copy(u_hbm.at[sv.at[j]], rows_v.at[k], gsem[k])
                pltpu.make_async_copy(u_hbm.at[sv.at[j]], rows_v.at[k], gsem[k]).wait()
                pltpu.async_copy(rows_v.at[k], acc_sh.at[dv.at[j]], ssem[k], add=True)
                if j < 2:
                    @pl.when(cc > 0)
                    def _():
                        pltpu.make_async_copy(
                            rows_v.at[k2], acc_sh.at[dv.at[j]], ssem[k2]).wait()
                else:
                    pltpu.make_async_copy(
                        rows_v.at[k2], acc_sh.at[dv.at[j]], ssem[k2]).wait()
                if j < cw - 2:
                    pltpu.async_copy(u_hbm.at[sv.at[j + 2]], rows_v.at[k2], gsem[k2])
                else:
                    @pl.when(cc + 1 < ncc)
                    def _():
                        pltpu.async_copy(
                            u_hbm.at[svn.at[j - (cw - 2)]], rows_v.at[k2], gsem[k2])
            return carry

        lax.fori_loop(0, ncc, chunk_body, 0)
        dvl = dst_cv.at[(ncc - 1) % 2]
        pltpu.make_async_copy(
            rows_v.at[(cw - 2) % rb], acc_sh.at[dvl.at[cw - 2]], ssem[(cw - 2) % rb]).wait()
        pltpu.make_async_copy(
            rows_v.at[(cw - 1) % rb], acc_sh.at[dvl.at[cw - 1]], ssem[(cw - 1) % rb]).wait()
        plsc.subcore_barrier()
        pltpu.sync_copy(acc_sh.at[pl.ds(s * rt, rt)], out_hbm.at[c, pl.ds(s * rt, rt)])

    return prop_kernel


def _dis_from_degp(degp_ref):
    deg = degp_ref[0, :, 0:1] + degp_ref[1, :, 0:1]
    safe = jnp.where(deg > 0.0, deg, 1.0)
    return jnp.where(deg > 0.0, lax.rsqrt(safe), 0.0)


def _tc_pre(degp, x, w1, n_pad, blk, d):
    def body(degp_ref, x_ref, w_ref, o_ref):
        dis = _dis_from_degp(degp_ref)
        o_ref[...] = jnp.dot(x_ref[...], w_ref[...],
                             preferred_element_type=jnp.float32) * dis

    return pl.pallas_call(
        body,
        grid=(n_pad // blk,),
        in_specs=[
            pl.BlockSpec((NC, blk, LANES), lambda i: (0, i, 0)),
            pl.BlockSpec((blk, d), lambda i: (i, 0)),
            pl.BlockSpec((d, d), lambda i: (0, 0)),
        ],
        out_specs=pl.BlockSpec((blk, d), lambda i: (i, 0)),
        out_shape=jax.ShapeDtypeStruct((n_pad, d), jnp.float32),
    )(degp, x, w1)


def _tc_mid(degp, x, s1p, w0, b, w1, n_pad, blk, d):
    def body(degp_ref, x_ref, sp_ref, w0_ref, b_ref, w1_ref, h_ref, u_ref):
        dis = _dis_from_degp(degp_ref)
        s_sum = sp_ref[0] + sp_ref[1]
        h = jnp.maximum(
            jnp.dot(x_ref[...], w0_ref[...], preferred_element_type=jnp.float32)
            - dis * s_sum + b_ref[...], 0.0)
        h_ref[...] = h
        u_ref[...] = jnp.dot(h, w1_ref[...],
                             preferred_element_type=jnp.float32) * dis

    return pl.pallas_call(
        body,
        grid=(n_pad // blk,),
        in_specs=[
            pl.BlockSpec((NC, blk, LANES), lambda i: (0, i, 0)),
            pl.BlockSpec((blk, d), lambda i: (i, 0)),
            pl.BlockSpec((NC, blk, d), lambda i: (0, i, 0)),
            pl.BlockSpec((d, d), lambda i: (0, 0)),
            pl.BlockSpec((1, d), lambda i: (0, 0)),
            pl.BlockSpec((d, d), lambda i: (0, 0)),
        ],
        out_specs=[
            pl.BlockSpec((blk, d), lambda i: (i, 0)),
            pl.BlockSpec((blk, d), lambda i: (i, 0)),
        ],
        out_shape=[
            jax.ShapeDtypeStruct((n_pad, d), jnp.float32),
            jax.ShapeDtypeStruct((n_pad, d), jnp.float32),
        ],
    )(degp, x, s1p, w0, b, w1)


def _tc_post(degp, h, s2p, w0, b, wfc_row, bfc2, n_pad, blk, d):
    def body(degp_ref, h_ref, sp_ref, w0_ref, b_ref, wfc_ref, bfc_ref,
             emb_ref, out_ref):
        dis = _dis_from_degp(degp_ref)
        s_sum = sp_ref[0] + sp_ref[1]
        emb = (jnp.dot(h_ref[...], w0_ref[...], preferred_element_type=jnp.float32)
               - dis * s_sum + b_ref[...])
        emb_ref[...] = emb
        logits = (jnp.sum(emb * wfc_ref[...], axis=1, keepdims=True)
                  + bfc_ref[:, 0:1])
        out_ref[...] = jax.nn.sigmoid(logits)

    return pl.pallas_call(
        body,
        grid=(n_pad // blk,),
        in_specs=[
            pl.BlockSpec((NC, blk, LANES), lambda i: (0, i, 0)),
            pl.BlockSpec((blk, d), lambda i: (i, 0)),
            pl.BlockSpec((NC, blk, d), lambda i: (0, i, 0)),
            pl.BlockSpec((d, d), lambda i: (0, 0)),
            pl.BlockSpec((1, d), lambda i: (0, 0)),
            pl.BlockSpec((1, d), lambda i: (0, 0)),
            pl.BlockSpec((1, 1), lambda i: (0, 0)),
        ],
        out_specs=[
            pl.BlockSpec((blk, d), lambda i: (i, 0)),
            pl.BlockSpec((blk, 1), lambda i: (i, 0)),
        ],
        out_shape=[
            jax.ShapeDtypeStruct((n_pad, d), jnp.float32),
            jax.ShapeDtypeStruct((n_pad, 1), jnp.float32),
        ],
    )(degp, h, s2p, w0, b, wfc_row, bfc2)


def kernel(x, edge_index, W0a, W1a, ba, W0b, W1b, bb, Wfc, bfc):
    n, d = x.shape
    e = edge_index.shape[1]
    bp = 80
    ch_deg = -(-e // (NW * B))
    ch = -(-e // (NW * bp))
    ch += (-ch) % 16
    e_pad_deg = NW * ch_deg * B
    e_pad = NW * ch * bp
    n_pad = -(-n // (NS * LANES)) * (NS * LANES)
    if n_pad < n + 8:
        n_pad += NS * LANES
    blk = max(b for b in range(8, min(n, 2048) + 1, 8) if n % b == 0)

    src = edge_index[0]
    dst = edge_index[1]
    pad = jnp.arange(e_pad - e, dtype=jnp.int32) % 8
    pad_d = jnp.arange(e_pad_deg - e, dtype=jnp.int32) % 8
    src_g = jnp.concatenate([src, pad]).reshape(NW, ch, bp)
    src_d = jnp.concatenate([src, n + pad_d]).reshape(NW, ch_deg, B)
    dst_p = jnp.concatenate([dst, n + pad]).reshape(NW, ch, bp)
    zeros_row = jnp.zeros((n_pad, d), jnp.float32)

    degp = _make_deg_kernel(n_pad, ch_deg)(src_d)

    prop = _make_prop_kernel(n_pad, ch, d)
    u1 = _tc_pre(degp, x, W1a, n, blk, d)
    s1p = prop(u1, src_g, dst_p, zeros_row)
    h, u2 = _tc_mid(degp, x, s1p, W0a, ba.reshape(1, d), W1b, n, blk, d)
    s2p = prop(u2, src_g, dst_p, zeros_row)
    emb, out = _tc_post(degp, h, s2p, W0b, bb.reshape(1, d),
                        Wfc.reshape(1, d), bfc.reshape(1, 1), n, blk, d)
    return (out, emb)
